# Initial kernel scaffold; baseline (speedup 1.0000x reference)
import jax
import jax.numpy as jnp
from jax.experimental import pallas as pl


def kernel(x, table):
    raise NotImplementedError("write your pallas kernel here")


# speedup vs baseline: 1.5354x; 1.5354x over previous
import functools

import numpy as np
import jax
import jax.numpy as jnp
from jax import lax
from jax.experimental import pallas as pl
from jax.experimental.pallas import tpu as pltpu
from jax.experimental.pallas import tpu_sc as plsc

D = 128
SCALE = float(np.sqrt(128.0))
NC, NS = 2, 16
NW = NC * NS
B_TOTAL = 4096 * 50
PER_W = B_TOTAL // NW
CHUNK = 128
NCH = PER_W // CHUNK
LPR = D // 16


def _emb_body(idx_hbm, table_hbm, out_hbm, idx_v, inb, outb, gsem):
    wid = lax.axis_index("s") * NC + lax.axis_index("c")
    base = wid * PER_W
    pltpu.sync_copy(idx_hbm.at[wid], idx_v)

    def chunk_body(j, carry):
        pltpu.async_copy(table_hbm.at[idx_v.at[j]], inb, gsem).wait()

        def row_body(r, c):
            for o in range(LPR):
                sl = pl.ds(o * 16, 16)
                outb[r, sl] = inb[r, sl] * SCALE
            return c

        lax.fori_loop(0, CHUNK, row_body, 0, unroll=2)

        pltpu.sync_copy(outb, out_hbm.at[pl.ds(base + j * CHUNK, CHUNK)])
        return carry

    lax.fori_loop(0, NCH, chunk_body, 0)


_emb_call = functools.partial(
    pl.kernel,
    out_type=jax.ShapeDtypeStruct((B_TOTAL, D), jnp.float32),
    mesh=plsc.VectorSubcoreMesh(core_axis_name="c", subcore_axis_name="s"),
    scratch_types=[
        pltpu.VMEM((NCH, CHUNK), jnp.int32),
        pltpu.VMEM((CHUNK, D), jnp.float32),
        pltpu.VMEM((CHUNK, D), jnp.float32),
        pltpu.SemaphoreType.DMA,
    ],
)(_emb_body)


def kernel(x, table):
    idx = x.reshape(NW, NCH, CHUNK).astype(jnp.int32)
    out = _emb_call(idx, table)
    return out.reshape(x.shape + (D,))
